# initial kernel scaffold (unmeasured)
import jax
import jax.numpy as jnp
from jax import lax
from jax.experimental import pallas as pl
from jax.experimental.pallas import tpu as pltpu


def kernel(
    x,
):
    def body(*refs):
        pass

    out_shape = jax.ShapeDtypeStruct(..., jnp.float32)
    return pl.pallas_call(body, out_shape=out_shape)(...)


# baseline (device time: 152888 ns/iter reference)
import jax
import jax.numpy as jnp
from jax import lax
from jax.experimental import pallas as pl
from jax.experimental.pallas import tpu as pltpu

N_Z = 4


def kernel(x):
    m_per, n = x.shape

    def body(x_ref, out_ref, comm_ref, send_sems, recv_sems):
        my_x = lax.axis_index("x")
        my_y = lax.axis_index("y")
        my_z = lax.axis_index("z")
        left = (my_z - 1) % N_Z
        right = (my_z + 1) % N_Z

        barrier_sem = pltpu.get_barrier_semaphore()
        for nbr in (left, right):
            pl.semaphore_signal(
                barrier_sem,
                inc=1,
                device_id=(my_x, my_y, nbr),
                device_id_type=pl.DeviceIdType.MESH,
            )
        pl.semaphore_wait(barrier_sem, 2)

        comm_ref[0, :, :] = x_ref[:, :]
        out_ref[pl.ds(my_z * m_per, m_per), :] = x_ref[:, :]

        for h in range(N_Z - 1):
            rdma = pltpu.make_async_remote_copy(
                src_ref=comm_ref.at[h],
                dst_ref=comm_ref.at[h + 1],
                send_sem=send_sems.at[h],
                recv_sem=recv_sems.at[h],
                device_id=(my_x, my_y, right),
                device_id_type=pl.DeviceIdType.MESH,
            )
            rdma.start()
            rdma.wait()
            origin = (my_z - h - 1) % N_Z
            out_ref[pl.ds(origin * m_per, m_per), :] = comm_ref[h + 1, :, :]

    return pl.pallas_call(
        body,
        out_shape=jax.ShapeDtypeStruct((N_Z * m_per, n), x.dtype),
        in_specs=[pl.BlockSpec(memory_space=pltpu.VMEM)],
        out_specs=pl.BlockSpec(memory_space=pltpu.VMEM),
        scratch_shapes=[
            pltpu.VMEM((N_Z, m_per, n), x.dtype),
            pltpu.SemaphoreType.DMA((N_Z - 1,)),
            pltpu.SemaphoreType.DMA((N_Z - 1,)),
        ],
        compiler_params=pltpu.CompilerParams(collective_id=0),
    )(x)


# device time: 108423 ns/iter; 1.4101x vs baseline; 1.4101x over previous
import jax
import jax.numpy as jnp
from jax import lax
from jax.experimental import pallas as pl
from jax.experimental.pallas import tpu as pltpu

N_Z = 4
N_S = N_Z - 1


def kernel(x):
    m_per, n = x.shape
    half = m_per // 2

    def body(
        x_ref,
        out_ref,
        zr_send,
        zr_recv,
        zl_send,
        zl_recv,
        xr_send,
        xr_recv,
        xl_send,
        xl_recv,
    ):
        my_x = lax.axis_index("x")
        my_y = lax.axis_index("y")
        my_z = lax.axis_index("z")
        xn = (1 - my_x, my_y, my_z)
        zr = (my_x, my_y, (my_z + 1) % N_Z)
        zl = (my_x, my_y, (my_z - 1) % N_Z)

        barrier = pltpu.get_barrier_semaphore()
        for nbr in (xn, zr, zl):
            pl.semaphore_signal(
                barrier, inc=1, device_id=nbr,
                device_id_type=pl.DeviceIdType.MESH,
            )
        pl.semaphore_wait(barrier, 3)

        def clamp(v):
            return jnp.clip(v, 0, N_Z - 1)

        def half_ref(ref, z_origin, xh):
            return ref.at[pl.ds(z_origin * m_per + xh * half, half), :]

        def rdma(src, dst, ssem, rsem, dev):
            return pltpu.make_async_remote_copy(
                src_ref=src, dst_ref=dst, send_sem=ssem, recv_sem=rsem,
                device_id=dev, device_id_type=pl.DeviceIdType.MESH,
            )

        def zr_send_el(s):
            return (my_z < N_Z - 1) & (s <= my_z)

        def zr_recv_el(s):
            return (my_z >= 1) & (s <= my_z - 1)

        def zl_send_el(s):
            return (my_z >= 1) & (s <= N_Z - 1 - my_z)

        def zl_recv_el(s):
            return (my_z <= N_Z - 2) & (s <= N_Z - 2 - my_z)

        def zr_send_org(s):
            return clamp(my_z - s)

        def zr_recv_org(s):
            return clamp(my_z - 1 - s)

        def zl_send_org(s):
            return clamp(my_z + s)

        def zl_recv_org(s):
            return clamp(my_z + 1 + s)

        def zr_rdma(s):
            org = zr_send_org(s)
            return rdma(half_ref(out_ref, org, my_x),
                        half_ref(out_ref, org, my_x),
                        zr_send.at[s], zr_recv.at[s], zr)

        def zl_rdma(s):
            org = zl_send_org(s)
            return rdma(half_ref(out_ref, org, my_x),
                        half_ref(out_ref, org, my_x),
                        zl_send.at[s], zl_recv.at[s], zl)

        def zr_recv_rdma(s):
            org = zr_recv_org(s)
            return rdma(half_ref(out_ref, org, my_x),
                        half_ref(out_ref, org, my_x),
                        zr_send.at[s], zr_recv.at[s], zl)

        def zl_recv_rdma(s):
            org = zl_recv_org(s)
            return rdma(half_ref(out_ref, org, my_x),
                        half_ref(out_ref, org, my_x),
                        zl_send.at[s], zl_recv.at[s], zr)

        def xr_rdma(s):
            org = zr_recv_org(s)
            return rdma(half_ref(out_ref, org, my_x),
                        half_ref(out_ref, org, my_x),
                        xr_send.at[s], xr_recv.at[s], xn)

        def xl_rdma(s):
            org = zl_recv_org(s)
            return rdma(half_ref(out_ref, org, my_x),
                        half_ref(out_ref, org, my_x),
                        xl_send.at[s], xl_recv.at[s], xn)

        def xr_in_rdma(s):
            org = zr_recv_org(s)
            return rdma(half_ref(out_ref, org, 1 - my_x),
                        half_ref(out_ref, org, 1 - my_x),
                        xr_send.at[s], xr_recv.at[s], xn)

        def xl_in_rdma(s):
            org = zl_recv_org(s)
            return rdma(half_ref(out_ref, org, 1 - my_x),
                        half_ref(out_ref, org, 1 - my_x),
                        xl_send.at[s], xl_recv.at[s], xn)

        out_ref[pl.ds(my_z * m_per, m_per), :] = x_ref[:, :]

        @pl.when(zr_send_el(0))
        def _():
            rdma(x_ref.at[pl.ds(my_x * half, half), :],
                 half_ref(out_ref, my_z, my_x),
                 zr_send.at[0], zr_recv.at[0], zr).start()

        @pl.when(zl_send_el(0))
        def _():
            rdma(x_ref.at[pl.ds(my_x * half, half), :],
                 half_ref(out_ref, my_z, my_x),
                 zl_send.at[0], zl_recv.at[0], zl).start()

        for s in range(1, N_S):
            @pl.when(zr_recv_el(s - 1))
            def _(s=s):
                zr_recv_rdma(s - 1).wait_recv()

            @pl.when(zr_send_el(s))
            def _(s=s):
                zr_rdma(s).start()

            @pl.when(zr_recv_el(s - 1))
            def _(s=s):
                xr_rdma(s - 1).start()

            @pl.when(zl_recv_el(s - 1))
            def _(s=s):
                zl_recv_rdma(s - 1).wait_recv()

            @pl.when(zl_send_el(s))
            def _(s=s):
                zl_rdma(s).start()

            @pl.when(zl_recv_el(s - 1))
            def _(s=s):
                xl_rdma(s - 1).start()

        @pl.when(zr_recv_el(N_S - 1))
        def _():
            zr_recv_rdma(N_S - 1).wait_recv()
            xr_rdma(N_S - 1).start()

        @pl.when(zl_recv_el(N_S - 1))
        def _():
            zl_recv_rdma(N_S - 1).wait_recv()
            xl_rdma(N_S - 1).start()

        for s in range(N_S):
            @pl.when(zr_recv_el(s))
            def _(s=s):
                xr_in_rdma(s).wait_recv()

            @pl.when(zl_recv_el(s))
            def _(s=s):
                xl_in_rdma(s).wait_recv()

        for s in range(N_S):
            @pl.when(zr_send_el(s))
            def _(s=s):
                zr_rdma(s).wait_send()

            @pl.when(zl_send_el(s))
            def _(s=s):
                zl_rdma(s).wait_send()

            @pl.when(zr_recv_el(s))
            def _(s=s):
                xr_rdma(s).wait_send()

            @pl.when(zl_recv_el(s))
            def _(s=s):
                xl_rdma(s).wait_send()

    return pl.pallas_call(
        body,
        out_shape=jax.ShapeDtypeStruct((N_Z * m_per, n), x.dtype),
        in_specs=[pl.BlockSpec(memory_space=pltpu.VMEM)],
        out_specs=pl.BlockSpec(memory_space=pltpu.VMEM),
        scratch_shapes=[
            pltpu.SemaphoreType.DMA((N_S,)),
            pltpu.SemaphoreType.DMA((N_S,)),
            pltpu.SemaphoreType.DMA((N_S,)),
            pltpu.SemaphoreType.DMA((N_S,)),
            pltpu.SemaphoreType.DMA((N_S,)),
            pltpu.SemaphoreType.DMA((N_S,)),
            pltpu.SemaphoreType.DMA((N_S,)),
            pltpu.SemaphoreType.DMA((N_S,)),
        ],
        compiler_params=pltpu.CompilerParams(collective_id=0),
    )(x)
